# baseline (speedup 1.0000x reference)
import functools

import jax
import jax.numpy as jnp
from jax.experimental import pallas as pl
from jax.experimental.pallas import tpu as pltpu

_CODEBOOK = 1024
_DIM = 256
_BETA = 0.25
_ROWS = 16384
_TOK = 1024
_B = 4
_T = 4


def _vq_block_kernel(z_ref, w_ref, z2_ref, w2_ref, zq_ref, idx_ref, loss_ref):
    b = pl.program_id(0)
    t = pl.program_id(1)
    zs = z_ref[0]
    w = w_ref[...]
    z2 = z2_ref[0]
    w2 = w2_ref[...]

    mmT = jax.lax.dot_general(
        w, zs, (((1,), (0,)), ((), ())),
        preferred_element_type=jnp.float32)

    dT = z2 + w2 - 2.0 * mmT

    dmin = jnp.min(dT, axis=0, keepdims=True)
    rows = jax.lax.broadcasted_iota(jnp.int32, (_CODEBOOK, _TOK), 0)
    idx = jnp.min(jnp.where(dT == dmin, rows, _CODEBOOK), axis=0)
    idx = idx.astype(jnp.int32)
    idx_ref[0, 0, :] = idx

    onehot = (rows == idx[None, :]).astype(jnp.float32)
    w_hi = w.astype(jnp.bfloat16).astype(jnp.float32)
    r1 = w - w_hi
    w_mid = r1.astype(jnp.bfloat16).astype(jnp.float32)
    w_lo = r1 - w_mid

    def _sel(part):
        return jax.lax.dot_general(
            part, onehot, (((0,), (0,)), ((), ())),
            preferred_element_type=jnp.float32)

    zqT = (_sel(w_hi) + _sel(w_mid)) + _sel(w_lo)

    diff = zqT - zs
    zq_ref[0] = zs + diff

    @pl.when((b == 0) & (t == 0))
    def _():
        loss_ref[...] = jnp.zeros((1, 1), jnp.float32)
    loss_ref[...] += jnp.sum(diff * diff).reshape(1, 1)


@functools.partial(jax.jit)
def kernel(z, W):
    B, C, T, H, Wd = z.shape
    z3 = z.reshape(B, C, T * H * Wd)

    z2x = (z ** 2).sum(axis=1).reshape(_ROWS, 1)
    w2x = (W ** 2).sum(axis=1)

    zq4, idx3, loss_sum = pl.pallas_call(
        _vq_block_kernel,
        grid=(_B, _T),
        in_specs=[
            pl.BlockSpec((1, _DIM, _TOK), lambda b, t: (b, 0, t)),
            pl.BlockSpec((_CODEBOOK, _DIM), lambda b, t: (0, 0)),
            pl.BlockSpec((1, 1, _TOK), lambda b, t: (b * _T + t, 0, 0)),
            pl.BlockSpec((_CODEBOOK, 1), lambda b, t: (0, 0)),
        ],
        out_specs=[
            pl.BlockSpec((1, _DIM, _TOK), lambda b, t: (b, 0, t)),
            pl.BlockSpec((1, 1, _TOK), lambda b, t: (b * _T + t, 0, 0)),
            pl.BlockSpec((1, 1), lambda b, t: (0, 0)),
        ],
        out_shape=[
            jax.ShapeDtypeStruct((B, _DIM, T * _TOK), jnp.float32),
            jax.ShapeDtypeStruct((_B * _T, 1, _TOK), jnp.int32),
            jax.ShapeDtypeStruct((1, 1), jnp.float32),
        ],
        compiler_params=pltpu.CompilerParams(
            dimension_semantics=("arbitrary", "arbitrary")),
    )(z3, W, z2x.reshape(_B * _T, 1, _TOK), w2x.reshape(_CODEBOOK, 1))

    d_argmin = idx3.reshape(_ROWS)
    mean_sq = loss_sum[0, 0] / (_ROWS * _DIM)
    loss = mean_sq + _BETA * mean_sq
    z_q = zq4.reshape(B, C, T, H, Wd)
    return (z_q, loss, d_argmin)


# speedup vs baseline: 1.0051x; 1.0051x over previous
import functools

import jax
import jax.numpy as jnp
from jax.experimental import pallas as pl
from jax.experimental.pallas import tpu as pltpu

_CODEBOOK = 1024
_DIM = 256
_BETA = 0.25
_ROWS = 16384
_TOK = 1024
_B = 4
_T = 4


def _vq_block_kernel(z_ref, w_ref, z2_ref, w2_ref, zq_ref, idx_ref, loss_ref):
    zs = z_ref[0]
    w = w_ref[...]
    z2 = z2_ref[0]
    w2 = w2_ref[...]

    mmT = jax.lax.dot_general(
        w, zs, (((1,), (0,)), ((), ())),
        preferred_element_type=jnp.float32)

    dT = z2 + w2 - 2.0 * mmT

    dmin = jnp.min(dT, axis=0, keepdims=True)
    rows = jax.lax.broadcasted_iota(jnp.int32, (_CODEBOOK, _TOK), 0)
    idx = jnp.min(jnp.where(dT == dmin, rows, _CODEBOOK), axis=0)
    idx = idx.astype(jnp.int32)
    idx_ref[0, 0, :] = idx

    onehot = (rows == idx[None, :]).astype(jnp.float32)
    w_hi = w.astype(jnp.bfloat16).astype(jnp.float32)
    r1 = w - w_hi
    w_mid = r1.astype(jnp.bfloat16).astype(jnp.float32)
    w_lo = r1 - w_mid

    def _sel(part):
        return jax.lax.dot_general(
            part, onehot, (((0,), (0,)), ((), ())),
            preferred_element_type=jnp.float32)

    zqT = (_sel(w_hi) + _sel(w_mid)) + _sel(w_lo)

    diff = zqT - zs
    zq_ref[0] = zs + diff

    loss_ref[...] = jnp.sum(diff * diff).reshape(1, 1, 1)


@functools.partial(jax.jit)
def kernel(z, W):
    B, C, T, H, Wd = z.shape
    z3 = z.reshape(B, C, T * H * Wd)

    z2x = (z ** 2).sum(axis=1).reshape(_ROWS, 1)
    w2x = (W ** 2).sum(axis=1)

    zq4, idx3, loss_sum = pl.pallas_call(
        _vq_block_kernel,
        grid=(_B, _T),
        in_specs=[
            pl.BlockSpec((1, _DIM, _TOK), lambda b, t: (b, 0, t)),
            pl.BlockSpec((_CODEBOOK, _DIM), lambda b, t: (0, 0)),
            pl.BlockSpec((1, 1, _TOK), lambda b, t: (b * _T + t, 0, 0)),
            pl.BlockSpec((_CODEBOOK, 1), lambda b, t: (0, 0)),
        ],
        out_specs=[
            pl.BlockSpec((1, _DIM, _TOK), lambda b, t: (b, 0, t)),
            pl.BlockSpec((1, 1, _TOK), lambda b, t: (b * _T + t, 0, 0)),
            pl.BlockSpec((1, 1, 1), lambda b, t: (b * _T + t, 0, 0)),
        ],
        out_shape=[
            jax.ShapeDtypeStruct((B, _DIM, T * _TOK), jnp.float32),
            jax.ShapeDtypeStruct((_B * _T, 1, _TOK), jnp.int32),
            jax.ShapeDtypeStruct((_B * _T, 1, 1), jnp.float32),
        ],
        compiler_params=pltpu.CompilerParams(
            dimension_semantics=("parallel", "parallel")),
    )(z3, W, z2x.reshape(_B * _T, 1, _TOK), w2x.reshape(_CODEBOOK, 1))

    d_argmin = idx3.reshape(_ROWS)
    mean_sq = loss_sum.sum() / (_ROWS * _DIM)
    loss = mean_sq + _BETA * mean_sq
    z_q = zq4.reshape(B, C, T, H, Wd)
    return (z_q, loss, d_argmin)


# speedup vs baseline: 1.0388x; 1.0335x over previous
import functools

import jax
import jax.numpy as jnp
from jax.experimental import pallas as pl
from jax.experimental.pallas import tpu as pltpu

_CODEBOOK = 1024
_DIM = 256
_BETA = 0.25
_ROWS = 16384
_TOK = 2048
_B = 4
_T = 4


def _vq_block_kernel(z_ref, w_ref, z2_ref, w2_ref, zq_ref, idx_ref, loss_ref):
    zs = z_ref[0]
    w = w_ref[...]
    z2 = z2_ref[0]
    w2 = w2_ref[...]

    mmT = jax.lax.dot_general(
        w, zs, (((1,), (0,)), ((), ())),
        preferred_element_type=jnp.float32)

    dT = z2 + w2 - 2.0 * mmT

    dmin = jnp.min(dT, axis=0, keepdims=True)
    rows = jax.lax.broadcasted_iota(jnp.int32, (_CODEBOOK, _TOK), 0)
    idx = jnp.min(jnp.where(dT == dmin, rows, _CODEBOOK), axis=0)
    idx = idx.astype(jnp.int32)
    idx_ref[0, 0, :] = idx

    onehot = (rows == idx[None, :]).astype(jnp.float32)
    w_hi = w.astype(jnp.bfloat16).astype(jnp.float32)
    r1 = w - w_hi
    w_mid = r1.astype(jnp.bfloat16).astype(jnp.float32)
    w_lo = r1 - w_mid

    def _sel(part):
        return jax.lax.dot_general(
            part, onehot, (((0,), (0,)), ((), ())),
            preferred_element_type=jnp.float32)

    zqT = (_sel(w_hi) + _sel(w_mid)) + _sel(w_lo)

    diff = zqT - zs
    zq_ref[0] = zs + diff

    loss_ref[...] = jnp.sum(diff * diff).reshape(1, 1, 1)


@functools.partial(jax.jit)
def kernel(z, W):
    B, C, T, H, Wd = z.shape
    z3 = z.reshape(B, C, T * H * Wd)

    z2x = (z ** 2).sum(axis=1).reshape(_ROWS, 1)
    w2x = (W ** 2).sum(axis=1)

    zq4, idx3, loss_sum = pl.pallas_call(
        _vq_block_kernel,
        grid=(_B, _T // 2),
        in_specs=[
            pl.BlockSpec((1, _DIM, _TOK), lambda b, t: (b, 0, t)),
            pl.BlockSpec((_CODEBOOK, _DIM), lambda b, t: (0, 0)),
            pl.BlockSpec((1, 1, _TOK), lambda b, t: (b * (_T // 2) + t, 0, 0)),
            pl.BlockSpec((_CODEBOOK, 1), lambda b, t: (0, 0)),
        ],
        out_specs=[
            pl.BlockSpec((1, _DIM, _TOK), lambda b, t: (b, 0, t)),
            pl.BlockSpec((1, 1, _TOK), lambda b, t: (b * (_T // 2) + t, 0, 0)),
            pl.BlockSpec((1, 1, 1), lambda b, t: (b * (_T // 2) + t, 0, 0)),
        ],
        out_shape=[
            jax.ShapeDtypeStruct((B, _DIM, T * H * Wd), jnp.float32),
            jax.ShapeDtypeStruct((_B * (_T // 2), 1, _TOK), jnp.int32),
            jax.ShapeDtypeStruct((_B * (_T // 2), 1, 1), jnp.float32),
        ],
        compiler_params=pltpu.CompilerParams(
            dimension_semantics=("parallel", "parallel")),
    )(z3, W, z2x.reshape(_B * (_T // 2), 1, _TOK), w2x.reshape(_CODEBOOK, 1))

    d_argmin = idx3.reshape(_ROWS)
    mean_sq = loss_sum.sum() / (_ROWS * _DIM)
    loss = mean_sq + _BETA * mean_sq
    z_q = zq4.reshape(B, C, T, H, Wd)
    return (z_q, loss, d_argmin)
